# baseline (speedup 1.0000x reference)
import functools

import jax
import jax.numpy as jnp
from jax import lax
from jax.experimental import pallas as pl
from jax.experimental.pallas import tpu as pltpu
from jax.experimental.pallas import tpu_sc as plsc

_T = 200
_V = 1000
_D = 128
_SAMPLING_RATIO = 0.2

_NC = 2
_NS = 16
_NW = _NC * _NS
_N_IDS = 64 * _T
_PER_W = _N_IDS // _NW
_CHUNKS = ((0, 104), (104, 104), (208, 104), (312, 88))


@functools.partial(
    pl.kernel,
    mesh=plsc.VectorSubcoreMesh(core_axis_name="c", subcore_axis_name="s"),
    out_type=jax.ShapeDtypeStruct((_N_IDS, _D), jnp.float32),
    scratch_types=[
        pltpu.VMEM((_PER_W,), jnp.int32),
        pltpu.VMEM((_PER_W, _D), jnp.float32),
        pltpu.SemaphoreType.DMA,
    ],
)
def _sc_gather(w_hbm, idx_hbm, out_hbm, idx_v, rows_v, sem):
    wid = lax.axis_index("s") * _NC + lax.axis_index("c")
    base = wid * _PER_W
    pltpu.sync_copy(idx_hbm.at[pl.ds(base, _PER_W)], idx_v)
    copies = [
        pltpu.async_copy(w_hbm.at[idx_v.at[pl.ds(off, sz)]],
                         rows_v.at[pl.ds(off, sz)], sem)
        for off, sz in _CHUNKS
    ]
    for c in copies:
        c.wait()
    pltpu.sync_copy(rows_v, out_hbm.at[pl.ds(base, _PER_W)])


def _tc_body(lens_ref, ig_ref, dec_ref, ys_ref, emb_ref, pa_ref, rc_ref, rr_ref,
             out1_ref, out2_ref, out3_ref, tg_ref, tn_ref, ts_ref, tr_ref):
    b = pl.program_id(0)
    L = lens_ref[b]
    ig = ig_ref[0]
    d = dec_ref[0]
    ys = ys_ref[0]
    emb = emb_ref[0]
    pa = pa_ref[0]
    rcol = rc_ref[0]
    rrow = rr_ref[0]

    viota = lax.broadcasted_iota(jnp.int32, (_T, _V), 1)
    mx = jnp.max(d, axis=1, keepdims=True)
    pred = jnp.min(jnp.where(d == mx, viota, _V), axis=1, keepdims=True)

    not_ignore = ys != ig
    same = (ys == pred) & not_ignore
    same_num = jnp.sum(same.astype(jnp.int32))
    eff = jnp.maximum(
        ((L.astype(jnp.float32) - same_num.astype(jnp.float32))
         * _SAMPLING_RATIO).astype(jnp.int32), 0)

    tio = lax.broadcasted_iota(jnp.int32, (_T, _T), 0)
    uio = lax.broadcasted_iota(jnp.int32, (_T, _T), 1)
    gt = (rrow > rcol) | ((rrow == rcol) & (uio < tio))
    validu = uio < L
    rank = jnp.sum((gt & validu).astype(jnp.int32), axis=1, keepdims=True)

    t2 = lax.broadcasted_iota(jnp.int32, (_T, 1), 0)
    tgt = t2 < L
    imask = (rank < eff) & tgt & not_ignore

    tgtf = tgt.astype(jnp.float32)
    out1_ref[0] = jnp.where(imask, emb, pa) * tgtf
    out2_ref[0] = emb * tgtf
    out3_ref[0] = pa * tgtf
    tg_ref[0] = tgt.astype(jnp.int32)

    num = jnp.sum(not_ignore.astype(jnp.int32))

    @pl.when(b == 0)
    def _init():
        tn_ref[0, 0] = 0
        ts_ref[0, 0] = 0
        tr_ref[0, 0] = 0

    tn_ref[0, 0] += num
    ts_ref[0, 0] += same_num
    tr_ref[0, 0] += eff


def kernel(decoder_out, ys_pad, ys_pad_lens, pred_acoustic_embeds, ignore_id, W):
    B, T = ys_pad.shape
    r = jax.random.uniform(jax.random.key(123), (B, T))
    rcol = r.reshape(B, T, 1)
    rrow = r.reshape(B, 1, T)
    ys_i32 = ys_pad.astype(jnp.int32)
    ys3 = ys_i32.reshape(B, T, 1)
    lens = ys_pad_lens.astype(jnp.int32)
    ig = jnp.asarray(ignore_id, jnp.int32).reshape(1)

    tgt = jnp.arange(T, dtype=jnp.int32)[None, :] < lens[:, None]
    idx = (ys_i32 * tgt).reshape(-1)
    emb = _sc_gather(W, idx).reshape(B, T, _D)

    out_shapes = (
        jax.ShapeDtypeStruct((B, T, _D), jnp.float32),
        jax.ShapeDtypeStruct((B, T, _D), jnp.float32),
        jax.ShapeDtypeStruct((B, T, _D), jnp.float32),
        jax.ShapeDtypeStruct((B, T, 1), jnp.int32),
        jax.ShapeDtypeStruct((1, 1), jnp.int32),
        jax.ShapeDtypeStruct((1, 1), jnp.int32),
        jax.ShapeDtypeStruct((1, 1), jnp.int32),
    )
    grid = (B,)
    smem = pltpu.SMEM
    in_specs = [
        pl.BlockSpec(memory_space=smem),
        pl.BlockSpec(memory_space=smem),
        pl.BlockSpec((1, T, _V), lambda b: (b, 0, 0)),
        pl.BlockSpec((1, T, 1), lambda b: (b, 0, 0)),
        pl.BlockSpec((1, T, _D), lambda b: (b, 0, 0)),
        pl.BlockSpec((1, T, _D), lambda b: (b, 0, 0)),
        pl.BlockSpec((1, T, 1), lambda b: (b, 0, 0)),
        pl.BlockSpec((1, 1, T), lambda b: (b, 0, 0)),
    ]
    out_specs = [
        pl.BlockSpec((1, T, _D), lambda b: (b, 0, 0)),
        pl.BlockSpec((1, T, _D), lambda b: (b, 0, 0)),
        pl.BlockSpec((1, T, _D), lambda b: (b, 0, 0)),
        pl.BlockSpec((1, T, 1), lambda b: (b, 0, 0)),
        pl.BlockSpec((1, 1), lambda b: (0, 0), memory_space=smem),
        pl.BlockSpec((1, 1), lambda b: (0, 0), memory_space=smem),
        pl.BlockSpec((1, 1), lambda b: (0, 0), memory_space=smem),
    ]
    o1, o2, o3, tg, tn, ts, tr = pl.pallas_call(
        _tc_body,
        grid=grid,
        in_specs=in_specs,
        out_specs=out_specs,
        out_shape=out_shapes,
    )(lens, ig, decoder_out, ys3, emb, pred_acoustic_embeds, rcol, rrow)

    tgt3 = tg.astype(jnp.bool_)
    return (o1, o2, o3, tgt3,
            tn.reshape(()), ts.reshape(()), tr.reshape(()))


# speedup vs baseline: 2.8090x; 2.8090x over previous
import functools

import jax
import jax.numpy as jnp
from jax import lax
from jax.experimental import pallas as pl
from jax.experimental.pallas import tpu as pltpu
from jax.experimental.pallas import tpu_sc as plsc

_T = 200
_V = 1000
_D = 128
_SAMPLING_RATIO = 0.2

_NC = 2
_NS = 16
_NW = _NC * _NS
_N_IDS = 64 * _T
_PER_W = _N_IDS // _NW
_CHUNKS = ((0, 104), (104, 104), (208, 104), (312, 88))


@functools.partial(
    pl.kernel,
    mesh=plsc.VectorSubcoreMesh(core_axis_name="c", subcore_axis_name="s"),
    out_type=jax.ShapeDtypeStruct((_N_IDS, _D), jnp.float32),
    scratch_types=[
        pltpu.VMEM((_PER_W,), jnp.int32),
        pltpu.VMEM((_PER_W, _D), jnp.float32),
        pltpu.VMEM_SHARED((_V, _D), jnp.float32),
        pltpu.SemaphoreType.DMA,
    ],
)
def _sc_gather(w_hbm, idx_hbm, out_hbm, idx_v, rows_v, w_sh, sem):
    sid = lax.axis_index("s")
    wid = sid * _NC + lax.axis_index("c")
    base = wid * _PER_W
    @pl.when(sid == 0)
    def _stage():
        pltpu.sync_copy(w_hbm, w_sh)

    pltpu.sync_copy(idx_hbm.at[pl.ds(base, _PER_W)], idx_v)
    plsc.subcore_barrier()
    copies = [
        pltpu.async_copy(w_sh.at[idx_v.at[pl.ds(off, sz)]],
                         rows_v.at[pl.ds(off, sz)], sem)
        for off, sz in _CHUNKS
    ]
    for c in copies:
        c.wait()
    pltpu.sync_copy(rows_v, out_hbm.at[pl.ds(base, _PER_W)])


def _tc_body(lens_ref, ig_ref, dec_ref, ys_ref, emb_ref, pa_ref, rc_ref, rr_ref,
             out1_ref, out2_ref, out3_ref, tg_ref, tn_ref, ts_ref, tr_ref):
    b = pl.program_id(0)
    L = lens_ref[b]
    ig = ig_ref[0]
    d = dec_ref[0]
    ys = ys_ref[0]
    emb = emb_ref[0]
    pa = pa_ref[0]
    rcol = rc_ref[0]
    rrow = rr_ref[0]

    viota = lax.broadcasted_iota(jnp.int32, (_T, _V), 1)
    mx = jnp.max(d, axis=1, keepdims=True)
    pred = jnp.min(jnp.where(d == mx, viota, _V), axis=1, keepdims=True)

    not_ignore = ys != ig
    same = (ys == pred) & not_ignore
    same_num = jnp.sum(same.astype(jnp.int32))
    eff = jnp.maximum(
        ((L.astype(jnp.float32) - same_num.astype(jnp.float32))
         * _SAMPLING_RATIO).astype(jnp.int32), 0)

    tio = lax.broadcasted_iota(jnp.int32, (_T, _T), 0)
    uio = lax.broadcasted_iota(jnp.int32, (_T, _T), 1)
    gt = (rrow > rcol) | ((rrow == rcol) & (uio < tio))
    validu = uio < L
    rank = jnp.sum((gt & validu).astype(jnp.int32), axis=1, keepdims=True)

    t2 = lax.broadcasted_iota(jnp.int32, (_T, 1), 0)
    tgt = t2 < L
    imask = (rank < eff) & tgt & not_ignore

    tgtf = tgt.astype(jnp.float32)
    out1_ref[0] = jnp.where(imask, emb, pa) * tgtf
    out2_ref[0] = emb * tgtf
    out3_ref[0] = pa * tgtf
    tg_ref[0] = tgt.astype(jnp.int32)

    num = jnp.sum(not_ignore.astype(jnp.int32))

    @pl.when(b == 0)
    def _init():
        tn_ref[0, 0] = 0
        ts_ref[0, 0] = 0
        tr_ref[0, 0] = 0

    tn_ref[0, 0] += num
    ts_ref[0, 0] += same_num
    tr_ref[0, 0] += eff


def kernel(decoder_out, ys_pad, ys_pad_lens, pred_acoustic_embeds, ignore_id, W):
    B, T = ys_pad.shape
    r = jax.random.uniform(jax.random.key(123), (B, T))
    rcol = r.reshape(B, T, 1)
    rrow = r.reshape(B, 1, T)
    ys_i32 = ys_pad.astype(jnp.int32)
    ys3 = ys_i32.reshape(B, T, 1)
    lens = ys_pad_lens.astype(jnp.int32)
    ig = jnp.asarray(ignore_id, jnp.int32).reshape(1)

    tgt = jnp.arange(T, dtype=jnp.int32)[None, :] < lens[:, None]
    idx = (ys_i32 * tgt).reshape(-1)
    emb = _sc_gather(W, idx).reshape(B, T, _D)

    out_shapes = (
        jax.ShapeDtypeStruct((B, T, _D), jnp.float32),
        jax.ShapeDtypeStruct((B, T, _D), jnp.float32),
        jax.ShapeDtypeStruct((B, T, _D), jnp.float32),
        jax.ShapeDtypeStruct((B, T, 1), jnp.int32),
        jax.ShapeDtypeStruct((1, 1), jnp.int32),
        jax.ShapeDtypeStruct((1, 1), jnp.int32),
        jax.ShapeDtypeStruct((1, 1), jnp.int32),
    )
    grid = (B,)
    smem = pltpu.SMEM
    in_specs = [
        pl.BlockSpec(memory_space=smem),
        pl.BlockSpec(memory_space=smem),
        pl.BlockSpec((1, T, _V), lambda b: (b, 0, 0)),
        pl.BlockSpec((1, T, 1), lambda b: (b, 0, 0)),
        pl.BlockSpec((1, T, _D), lambda b: (b, 0, 0)),
        pl.BlockSpec((1, T, _D), lambda b: (b, 0, 0)),
        pl.BlockSpec((1, T, 1), lambda b: (b, 0, 0)),
        pl.BlockSpec((1, 1, T), lambda b: (b, 0, 0)),
    ]
    out_specs = [
        pl.BlockSpec((1, T, _D), lambda b: (b, 0, 0)),
        pl.BlockSpec((1, T, _D), lambda b: (b, 0, 0)),
        pl.BlockSpec((1, T, _D), lambda b: (b, 0, 0)),
        pl.BlockSpec((1, T, 1), lambda b: (b, 0, 0)),
        pl.BlockSpec((1, 1), lambda b: (0, 0), memory_space=smem),
        pl.BlockSpec((1, 1), lambda b: (0, 0), memory_space=smem),
        pl.BlockSpec((1, 1), lambda b: (0, 0), memory_space=smem),
    ]
    o1, o2, o3, tg, tn, ts, tr = pl.pallas_call(
        _tc_body,
        grid=grid,
        in_specs=in_specs,
        out_specs=out_specs,
        out_shape=out_shapes,
    )(lens, ig, decoder_out, ys3, emb, pred_acoustic_embeds, rcol, rrow)

    tgt3 = tg.astype(jnp.bool_)
    return (o1, o2, o3, tgt3,
            tn.reshape(()), ts.reshape(()), tr.reshape(()))


# speedup vs baseline: 3.4505x; 1.2284x over previous
import functools

import jax
import jax.numpy as jnp
from jax import lax
from jax.experimental import pallas as pl
from jax.experimental.pallas import tpu as pltpu
from jax.experimental.pallas import tpu_sc as plsc

_T = 200
_V = 1000
_D = 128
_SAMPLING_RATIO = 0.2
_ROWS = 4

_NC = 2
_NS = 16
_NW = _NC * _NS
_N_IDS = 64 * _T
_PER_W = _N_IDS // _NW
_CHUNKS = ((0, 104), (104, 104), (208, 104), (312, 88))


@functools.lru_cache(maxsize=1)
def _sc_gather_fn():
    @functools.partial(
        pl.kernel,
        mesh=plsc.VectorSubcoreMesh(core_axis_name="c", subcore_axis_name="s"),
        out_type=jax.ShapeDtypeStruct((_N_IDS, _D), jnp.float32),
        scratch_types=[
            pltpu.VMEM((_PER_W,), jnp.int32),
            pltpu.VMEM((_PER_W, _D), jnp.float32),
            pltpu.VMEM_SHARED((_V, _D), jnp.float32),
            pltpu.SemaphoreType.DMA,
        ],
    )
    def _sc_gather(w_hbm, idx_hbm, out_hbm, idx_v, rows_v, w_sh, sem):
        sid = lax.axis_index("s")
        wid = sid * _NC + lax.axis_index("c")
        base = wid * _PER_W
        @pl.when(sid == 0)
        def _stage():
            pltpu.sync_copy(w_hbm, w_sh)

        pltpu.sync_copy(idx_hbm.at[pl.ds(base, _PER_W)], idx_v)
        plsc.subcore_barrier()
        copies = [
            pltpu.async_copy(w_sh.at[idx_v.at[pl.ds(off, sz)]],
                             rows_v.at[pl.ds(off, sz)], sem)
            for off, sz in _CHUNKS
        ]
        for c in copies:
            c.wait()
        pltpu.sync_copy(rows_v, out_hbm.at[pl.ds(base, _PER_W)])

    return _sc_gather


def _tc_body(lens_ref, ig_ref, dec_ref, ys_ref, emb_ref, pa_ref, rc_ref, rr_ref,
             out1_ref, out2_ref, out3_ref, tg_ref, tn_ref, ts_ref, tr_ref):
    g = pl.program_id(0)
    ig = ig_ref[0]

    @pl.when(g == 0)
    def _init():
        tn_ref[0, 0] = 0
        ts_ref[0, 0] = 0
        tr_ref[0, 0] = 0

    for rr in range(_ROWS):
        L = lens_ref[g * _ROWS + rr]
        d = dec_ref[rr]
        ys = ys_ref[rr]
        emb = emb_ref[rr]
        pa = pa_ref[rr]
        rcol = rc_ref[rr]
        rrow = rr_ref[rr]

        viota = lax.broadcasted_iota(jnp.int32, (_T, _V), 1)
        mx = jnp.max(d, axis=1, keepdims=True)
        pred = jnp.min(jnp.where(d == mx, viota, _V), axis=1, keepdims=True)

        not_ignore = ys != ig
        same = (ys == pred) & not_ignore
        same_num = jnp.sum(same.astype(jnp.int32))
        eff = jnp.maximum(
            ((L.astype(jnp.float32) - same_num.astype(jnp.float32))
             * _SAMPLING_RATIO).astype(jnp.int32), 0)

        tio = lax.broadcasted_iota(jnp.int32, (_T, _T), 0)
        uio = lax.broadcasted_iota(jnp.int32, (_T, _T), 1)
        gt = (rrow > rcol) | ((rrow == rcol) & (uio < tio))
        validu = uio < L
        rank = jnp.sum((gt & validu).astype(jnp.int32), axis=1, keepdims=True)

        t2 = lax.broadcasted_iota(jnp.int32, (_T, 1), 0)
        tgt = t2 < L
        imask = (rank < eff) & tgt & not_ignore

        tgtf = tgt.astype(jnp.float32)
        out1_ref[rr] = jnp.where(imask, emb, pa) * tgtf
        out2_ref[rr] = emb * tgtf
        out3_ref[rr] = pa * tgtf
        tg_ref[rr] = tgt.astype(jnp.int32)

        num = jnp.sum(not_ignore.astype(jnp.int32))
        tn_ref[0, 0] += num
        ts_ref[0, 0] += same_num
        tr_ref[0, 0] += eff


def kernel(decoder_out, ys_pad, ys_pad_lens, pred_acoustic_embeds, ignore_id, W):
    B, T = ys_pad.shape
    r = jax.random.uniform(jax.random.key(123), (B, T))
    rcol = r.reshape(B, T, 1)
    rrow = r.reshape(B, 1, T)
    ys_i32 = ys_pad.astype(jnp.int32)
    ys3 = ys_i32.reshape(B, T, 1)
    lens = ys_pad_lens.astype(jnp.int32)
    ig = jnp.asarray(ignore_id, jnp.int32).reshape(1)

    tgt = jnp.arange(T, dtype=jnp.int32)[None, :] < lens[:, None]
    idx = (ys_i32 * tgt).reshape(-1)
    emb = _sc_gather_fn()(W, idx).reshape(B, T, _D)

    out_shapes = (
        jax.ShapeDtypeStruct((B, T, _D), jnp.float32),
        jax.ShapeDtypeStruct((B, T, _D), jnp.float32),
        jax.ShapeDtypeStruct((B, T, _D), jnp.float32),
        jax.ShapeDtypeStruct((B, T, 1), jnp.int32),
        jax.ShapeDtypeStruct((1, 1), jnp.int32),
        jax.ShapeDtypeStruct((1, 1), jnp.int32),
        jax.ShapeDtypeStruct((1, 1), jnp.int32),
    )
    grid = (B // _ROWS,)
    smem = pltpu.SMEM
    in_specs = [
        pl.BlockSpec(memory_space=smem),
        pl.BlockSpec(memory_space=smem),
        pl.BlockSpec((_ROWS, T, _V), lambda g: (g, 0, 0)),
        pl.BlockSpec((_ROWS, T, 1), lambda g: (g, 0, 0)),
        pl.BlockSpec((_ROWS, T, _D), lambda g: (g, 0, 0)),
        pl.BlockSpec((_ROWS, T, _D), lambda g: (g, 0, 0)),
        pl.BlockSpec((_ROWS, T, 1), lambda g: (g, 0, 0)),
        pl.BlockSpec((_ROWS, 1, T), lambda g: (g, 0, 0)),
    ]
    out_specs = [
        pl.BlockSpec((_ROWS, T, _D), lambda g: (g, 0, 0)),
        pl.BlockSpec((_ROWS, T, _D), lambda g: (g, 0, 0)),
        pl.BlockSpec((_ROWS, T, _D), lambda g: (g, 0, 0)),
        pl.BlockSpec((_ROWS, T, 1), lambda g: (g, 0, 0)),
        pl.BlockSpec((1, 1), lambda g: (0, 0), memory_space=smem),
        pl.BlockSpec((1, 1), lambda g: (0, 0), memory_space=smem),
        pl.BlockSpec((1, 1), lambda g: (0, 0), memory_space=smem),
    ]
    o1, o2, o3, tg, tn, ts, tr = pl.pallas_call(
        _tc_body,
        grid=grid,
        in_specs=in_specs,
        out_specs=out_specs,
        out_shape=out_shapes,
    )(lens, ig, decoder_out, ys3, emb, pred_acoustic_embeds, rcol, rrow)

    tgt3 = tg.astype(jnp.bool_)
    return (o1, o2, o3, tgt3,
            tn.reshape(()), ts.reshape(()), tr.reshape(()))


# speedup vs baseline: 3.5264x; 1.0220x over previous
import functools

import jax
import jax.numpy as jnp
from jax import lax
from jax.experimental import pallas as pl
from jax.experimental.pallas import tpu as pltpu
from jax.experimental.pallas import tpu_sc as plsc

_T = 200
_V = 1000
_VP = 1008
_D = 128
_SAMPLING_RATIO = 0.2
_ROWS = 8

_NC = 2
_NS = 16
_NW = _NC * _NS
_N_IDS = 64 * _T
_PER_W = _N_IDS // _NW
_CHUNKS = ((0, 104), (104, 104), (208, 104), (312, 88))


@functools.lru_cache(maxsize=1)
def _sc_gather_fn():
    @functools.partial(
        pl.kernel,
        mesh=plsc.VectorSubcoreMesh(core_axis_name="c", subcore_axis_name="s"),
        out_type=jax.ShapeDtypeStruct((_N_IDS, _D), jnp.float32),
        scratch_types=[
            pltpu.VMEM((_PER_W,), jnp.int32),
            pltpu.VMEM((_PER_W, _D), jnp.float32),
            pltpu.VMEM_SHARED((_VP, _D), jnp.float32),
            pltpu.SemaphoreType.DMA,
        ],
    )
    def _sc_gather(w_hbm, idx_hbm, out_hbm, idx_v, rows_v, w_sh, sem):
        sid = lax.axis_index("s")
        wid = sid * _NC + lax.axis_index("c")
        base = wid * _PER_W
        @pl.when(sid == 0)
        def _stage():
            pltpu.sync_copy(w_hbm, w_sh)

        pltpu.sync_copy(idx_hbm.at[pl.ds(base, _PER_W)], idx_v)
        plsc.subcore_barrier()
        copies = [
            pltpu.async_copy(w_sh.at[idx_v.at[pl.ds(off, sz)]],
                             rows_v.at[pl.ds(off, sz)], sem)
            for off, sz in _CHUNKS
        ]
        for c in copies:
            c.wait()
        pltpu.sync_copy(rows_v, out_hbm.at[pl.ds(base, _PER_W)])

    return _sc_gather


def _tc_body(lens_ref, ig_ref, dec_ref, ys_ref, emb_ref, pa_ref, rc_ref, rr_ref,
             out1_ref, out3_ref, tg_ref, tn_ref, ts_ref, tr_ref):
    g = pl.program_id(0)
    ig = ig_ref[0]

    @pl.when(g == 0)
    def _init():
        tn_ref[0, 0] = 0
        ts_ref[0, 0] = 0
        tr_ref[0, 0] = 0

    for rr in range(_ROWS):
        L = lens_ref[g * _ROWS + rr]
        d = dec_ref[rr]
        ys = ys_ref[rr]
        emb = emb_ref[rr]
        pa = pa_ref[rr]
        rcol = rc_ref[rr]
        rrow = rr_ref[rr]

        viota = lax.broadcasted_iota(jnp.int32, (_T, _V), 1)
        mx = jnp.max(d, axis=1, keepdims=True)
        pred = jnp.min(jnp.where(d == mx, viota, _V), axis=1, keepdims=True)

        not_ignore = ys != ig
        same = (ys == pred) & not_ignore
        same_num = jnp.sum(same.astype(jnp.int32))
        eff = jnp.maximum(
            ((L.astype(jnp.float32) - same_num.astype(jnp.float32))
             * _SAMPLING_RATIO).astype(jnp.int32), 0)

        tio = lax.broadcasted_iota(jnp.int32, (_T, _T), 0)
        uio = lax.broadcasted_iota(jnp.int32, (_T, _T), 1)
        gt = (rrow > rcol) | ((rrow == rcol) & (uio < tio))
        validu = uio < L
        rank = jnp.sum((gt & validu).astype(jnp.int32), axis=1, keepdims=True)

        t2 = lax.broadcasted_iota(jnp.int32, (_T, 1), 0)
        tgt = t2 < L
        imask = (rank < eff) & tgt & not_ignore

        tgtf = tgt.astype(jnp.float32)
        out1_ref[rr] = jnp.where(imask, emb, pa * tgtf)
        out3_ref[rr] = pa * tgtf
        tg_ref[rr] = tgt.astype(jnp.int32)

        num = jnp.sum(not_ignore.astype(jnp.int32))
        tn_ref[0, 0] += num
        ts_ref[0, 0] += same_num
        tr_ref[0, 0] += eff


def kernel(decoder_out, ys_pad, ys_pad_lens, pred_acoustic_embeds, ignore_id, W):
    B, T = ys_pad.shape
    r = jax.random.uniform(jax.random.key(123), (B, T))
    rcol = r.reshape(B, T, 1)
    rrow = r.reshape(B, 1, T)
    ys_i32 = ys_pad.astype(jnp.int32)
    ys3 = ys_i32.reshape(B, T, 1)
    lens = ys_pad_lens.astype(jnp.int32)
    ig = jnp.asarray(ignore_id, jnp.int32).reshape(1)

    tgt = jnp.arange(T, dtype=jnp.int32)[None, :] < lens[:, None]
    idx = jnp.where(tgt, ys_i32, _V).reshape(-1)
    w_pad = jnp.concatenate([W, jnp.zeros((_VP - _V, _D), jnp.float32)], axis=0)
    out2_flat = _sc_gather_fn()(w_pad, idx)
    out2 = out2_flat.reshape(B, T, _D)

    out_shapes = (
        jax.ShapeDtypeStruct((B, T, _D), jnp.float32),
        jax.ShapeDtypeStruct((B, T, _D), jnp.float32),
        jax.ShapeDtypeStruct((B, T, 1), jnp.int32),
        jax.ShapeDtypeStruct((1, 1), jnp.int32),
        jax.ShapeDtypeStruct((1, 1), jnp.int32),
        jax.ShapeDtypeStruct((1, 1), jnp.int32),
    )
    grid = (B // _ROWS,)
    smem = pltpu.SMEM
    in_specs = [
        pl.BlockSpec(memory_space=smem),
        pl.BlockSpec(memory_space=smem),
        pl.BlockSpec((_ROWS, T, _V), lambda g: (g, 0, 0)),
        pl.BlockSpec((_ROWS, T, 1), lambda g: (g, 0, 0)),
        pl.BlockSpec((_ROWS, T, _D), lambda g: (g, 0, 0)),
        pl.BlockSpec((_ROWS, T, _D), lambda g: (g, 0, 0)),
        pl.BlockSpec((_ROWS, T, 1), lambda g: (g, 0, 0)),
        pl.BlockSpec((_ROWS, 1, T), lambda g: (g, 0, 0)),
    ]
    out_specs = [
        pl.BlockSpec((_ROWS, T, _D), lambda g: (g, 0, 0)),
        pl.BlockSpec((_ROWS, T, _D), lambda g: (g, 0, 0)),
        pl.BlockSpec((_ROWS, T, 1), lambda g: (g, 0, 0)),
        pl.BlockSpec((1, 1), lambda g: (0, 0), memory_space=smem),
        pl.BlockSpec((1, 1), lambda g: (0, 0), memory_space=smem),
        pl.BlockSpec((1, 1), lambda g: (0, 0), memory_space=smem),
    ]
    o1, o3, tg, tn, ts, tr = pl.pallas_call(
        _tc_body,
        grid=grid,
        in_specs=in_specs,
        out_specs=out_specs,
        out_shape=out_shapes,
    )(lens, ig, decoder_out, ys3, out2, pred_acoustic_embeds, rcol, rrow)

    tgt3 = tg.astype(jnp.bool_)
    return (o1, out2, o3, tgt3,
            tn.reshape(()), ts.reshape(()), tr.reshape(()))


# speedup vs baseline: 3.6563x; 1.0368x over previous
import functools

import jax
import jax.numpy as jnp
from jax import lax
from jax.experimental import pallas as pl
from jax.experimental.pallas import tpu as pltpu
from jax.experimental.pallas import tpu_sc as plsc

_T = 200
_V = 1000
_VP = 1008
_D = 128
_SAMPLING_RATIO = 0.2
_ROWS = 8

_NC = 2
_NS = 16
_NW = _NC * _NS
_N_IDS = 64 * _T
_PER_W = _N_IDS // _NW
_CHUNKS = ((0, 104), (104, 104), (208, 104), (312, 88))


@functools.lru_cache(maxsize=1)
def _sc_gather_fn():
    @functools.partial(
        pl.kernel,
        mesh=plsc.VectorSubcoreMesh(core_axis_name="c", subcore_axis_name="s"),
        out_type=jax.ShapeDtypeStruct((_N_IDS, _D), jnp.float32),
        scratch_types=[
            pltpu.VMEM((_PER_W,), jnp.int32),
            pltpu.VMEM((_PER_W, _D), jnp.float32),
            pltpu.VMEM_SHARED((_VP, _D), jnp.float32),
            pltpu.SemaphoreType.DMA,
        ],
    )
    def _sc_gather(w_hbm, idx_hbm, out_hbm, idx_v, rows_v, w_sh, sem):
        sid = lax.axis_index("s")
        wid = sid * _NC + lax.axis_index("c")
        base = wid * _PER_W
        @pl.when(sid == 0)
        def _stage():
            pltpu.sync_copy(w_hbm, w_sh)

        pltpu.sync_copy(idx_hbm.at[pl.ds(base, _PER_W)], idx_v)
        plsc.subcore_barrier()
        copies = [
            pltpu.async_copy(w_sh.at[idx_v.at[pl.ds(off, sz)]],
                             rows_v.at[pl.ds(off, sz)], sem)
            for off, sz in _CHUNKS
        ]
        for c in copies:
            c.wait()
        pltpu.sync_copy(rows_v, out_hbm.at[pl.ds(base, _PER_W)])

    return _sc_gather


def _tc_body(lens_ref, ig_ref, dec_ref, ys_ref, emb_ref, pa_ref, rc_ref, rr_ref,
             out1_ref, out3_ref, tg_ref, tn_ref, ts_ref, tr_ref):
    g = pl.program_id(0)
    ig = ig_ref[0]
    viota = lax.broadcasted_iota(jnp.int32, (_T, _V), 1).astype(jnp.float32)
    t2 = lax.broadcasted_iota(jnp.int32, (_T, 1), 0)

    @pl.when(g == 0)
    def _init():
        tn_ref[0, 0] = 0
        ts_ref[0, 0] = 0
        tr_ref[0, 0] = 0

    for rr in range(_ROWS):
        L = lens_ref[g * _ROWS + rr]
        d = dec_ref[rr]
        ys = ys_ref[rr]
        emb = emb_ref[rr]
        pa = pa_ref[rr]
        rcol = rc_ref[rr]
        rrow = rr_ref[rr]

        mx = jnp.max(d, axis=1, keepdims=True)
        predf = jnp.min(jnp.where(d == mx, viota, 2048.0), axis=1,
                        keepdims=True)
        pred = predf.astype(jnp.int32)

        not_ignore = ys != ig
        same = (ys == pred) & not_ignore
        same_num = jnp.sum(same.astype(jnp.int32))
        eff = jnp.maximum(
            ((L.astype(jnp.float32) - same_num.astype(jnp.float32))
             * _SAMPLING_RATIO).astype(jnp.int32), 0)

        tgt = t2 < L
        tgtf = tgt.astype(jnp.float32)

        gtf = (rrow > rcol).astype(jnp.float32)
        rank = lax.dot_general(gtf, tgtf, (((1,), (0,)), ((), ())),
                               preferred_element_type=jnp.float32)
        imask = (rank < eff.astype(jnp.float32)) & tgt & not_ignore

        out1_ref[rr] = jnp.where(imask, emb, pa * tgtf)
        out3_ref[rr] = pa * tgtf
        tg_ref[rr] = tgt.astype(jnp.int32)

        num = jnp.sum(not_ignore.astype(jnp.int32))
        tn_ref[0, 0] += num
        ts_ref[0, 0] += same_num
        tr_ref[0, 0] += eff


def kernel(decoder_out, ys_pad, ys_pad_lens, pred_acoustic_embeds, ignore_id, W):
    B, T = ys_pad.shape
    r = jax.random.uniform(jax.random.key(123), (B, T))
    rcol = r.reshape(B, T, 1)
    rrow = r.reshape(B, 1, T)
    ys_i32 = ys_pad.astype(jnp.int32)
    ys3 = ys_i32.reshape(B, T, 1)
    lens = ys_pad_lens.astype(jnp.int32)
    ig = jnp.asarray(ignore_id, jnp.int32).reshape(1)

    tgt = jnp.arange(T, dtype=jnp.int32)[None, :] < lens[:, None]
    idx = jnp.where(tgt, ys_i32, _V).reshape(-1)
    w_pad = jnp.concatenate([W, jnp.zeros((_VP - _V, _D), jnp.float32)], axis=0)
    out2_flat = _sc_gather_fn()(w_pad, idx)
    out2 = out2_flat.reshape(B, T, _D)

    out_shapes = (
        jax.ShapeDtypeStruct((B, T, _D), jnp.float32),
        jax.ShapeDtypeStruct((B, T, _D), jnp.float32),
        jax.ShapeDtypeStruct((B, T, 1), jnp.int32),
        jax.ShapeDtypeStruct((1, 1), jnp.int32),
        jax.ShapeDtypeStruct((1, 1), jnp.int32),
        jax.ShapeDtypeStruct((1, 1), jnp.int32),
    )
    grid = (B // _ROWS,)
    smem = pltpu.SMEM
    in_specs = [
        pl.BlockSpec(memory_space=smem),
        pl.BlockSpec(memory_space=smem),
        pl.BlockSpec((_ROWS, T, _V), lambda g: (g, 0, 0)),
        pl.BlockSpec((_ROWS, T, 1), lambda g: (g, 0, 0)),
        pl.BlockSpec((_ROWS, T, _D), lambda g: (g, 0, 0)),
        pl.BlockSpec((_ROWS, T, _D), lambda g: (g, 0, 0)),
        pl.BlockSpec((_ROWS, T, 1), lambda g: (g, 0, 0)),
        pl.BlockSpec((_ROWS, 1, T), lambda g: (g, 0, 0)),
    ]
    out_specs = [
        pl.BlockSpec((_ROWS, T, _D), lambda g: (g, 0, 0)),
        pl.BlockSpec((_ROWS, T, _D), lambda g: (g, 0, 0)),
        pl.BlockSpec((_ROWS, T, 1), lambda g: (g, 0, 0)),
        pl.BlockSpec((1, 1), lambda g: (0, 0), memory_space=smem),
        pl.BlockSpec((1, 1), lambda g: (0, 0), memory_space=smem),
        pl.BlockSpec((1, 1), lambda g: (0, 0), memory_space=smem),
    ]
    o1, o3, tg, tn, ts, tr = pl.pallas_call(
        _tc_body,
        grid=grid,
        in_specs=in_specs,
        out_specs=out_specs,
        out_shape=out_shapes,
    )(lens, ig, decoder_out, ys3, out2, pred_acoustic_embeds, rcol, rrow)

    tgt3 = tg.astype(jnp.bool_)
    return (o1, out2, o3, tgt3,
            tn.reshape(()), ts.reshape(()), tr.reshape(()))


# speedup vs baseline: 5.6461x; 1.5442x over previous
import functools

import jax
import jax.numpy as jnp
import numpy as np
from jax import lax
from jax.experimental import pallas as pl
from jax.experimental.pallas import tpu as pltpu
from jax.experimental.pallas import tpu_sc as plsc

_T = 200
_V = 1000
_VP = 1008
_D = 128
_SAMPLING_RATIO = 0.2
_ROWS = 8

_NC = 2
_NS = 16
_NW = _NC * _NS
_N_IDS = 64 * _T
_PER_W = _N_IDS // _NW
_CHUNKS = ((0, 104), (104, 104), (208, 104), (312, 88))

_R_CONST = np.asarray(jax.random.uniform(jax.random.key(123), (64, _T)))


@functools.lru_cache(maxsize=1)
def _sc_gather_fn():
    @functools.partial(
        pl.kernel,
        mesh=plsc.VectorSubcoreMesh(core_axis_name="c", subcore_axis_name="s"),
        out_type=jax.ShapeDtypeStruct((_N_IDS, _D), jnp.float32),
        scratch_types=[
            pltpu.VMEM((_PER_W,), jnp.int32),
            pltpu.VMEM((_PER_W, _D), jnp.float32),
            pltpu.VMEM((_VP - _V, _D), jnp.float32),
            pltpu.VMEM_SHARED((_VP, _D), jnp.float32),
            pltpu.SemaphoreType.DMA,
        ],
    )
    def _sc_gather(w_hbm, idx_hbm, out_hbm, idx_v, rows_v, zpad_v, w_sh, sem):
        sid = lax.axis_index("s")
        wid = sid * _NC + lax.axis_index("c")
        base = wid * _PER_W
        @pl.when(sid == 0)
        def _stage():
            for i in range(_VP - _V):
                for j in range(_D // 16):
                    zpad_v[i, pl.ds(j * 16, 16)] = jnp.zeros((16,), jnp.float32)
            pltpu.sync_copy(w_hbm, w_sh.at[pl.ds(0, _V)])
            pltpu.sync_copy(zpad_v, w_sh.at[pl.ds(_V, _VP - _V)])

        pltpu.sync_copy(idx_hbm.at[pl.ds(base, _PER_W)], idx_v)
        plsc.subcore_barrier()
        copies = [
            pltpu.async_copy(w_sh.at[idx_v.at[pl.ds(off, sz)]],
                             rows_v.at[pl.ds(off, sz)], sem)
            for off, sz in _CHUNKS
        ]
        for c in copies:
            c.wait()
        pltpu.sync_copy(rows_v, out_hbm.at[pl.ds(base, _PER_W)])

    return _sc_gather


def _tc_body(lens_ref, ig_ref, dec_ref, ys_ref, emb_ref, pa_ref, rc_ref, rr_ref,
             out1_ref, out3_ref, tn_ref, ts_ref, tr_ref):
    g = pl.program_id(0)
    ig = ig_ref[0]
    viota = lax.broadcasted_iota(jnp.int32, (_T, _V), 1).astype(jnp.float32)
    t2 = lax.broadcasted_iota(jnp.int32, (_T, 1), 0)

    @pl.when(g == 0)
    def _init():
        tn_ref[0, 0] = 0
        ts_ref[0, 0] = 0
        tr_ref[0, 0] = 0

    for rr in range(_ROWS):
        L = lens_ref[g * _ROWS + rr]
        d = dec_ref[rr]
        ys = ys_ref[rr]
        emb = emb_ref[rr]
        pa = pa_ref[rr]
        rcol = rc_ref[rr]
        rrow = rr_ref[rr]

        mx = jnp.max(d, axis=1, keepdims=True)
        predf = jnp.min(jnp.where(d == mx, viota, 2048.0), axis=1,
                        keepdims=True)
        pred = predf.astype(jnp.int32)

        not_ignore = ys != ig
        same = (ys == pred) & not_ignore
        same_num = jnp.sum(same.astype(jnp.int32))
        eff = jnp.maximum(
            ((L.astype(jnp.float32) - same_num.astype(jnp.float32))
             * _SAMPLING_RATIO).astype(jnp.int32), 0)

        tgt = t2 < L
        tgtf = tgt.astype(jnp.float32)

        gtf = (rrow > rcol).astype(jnp.float32)
        rank = lax.dot_general(gtf, tgtf, (((1,), (0,)), ((), ())),
                               preferred_element_type=jnp.float32)
        imask = (rank < eff.astype(jnp.float32)) & tgt & not_ignore

        out1_ref[rr] = jnp.where(imask, emb, pa * tgtf)
        out3_ref[rr] = pa * tgtf

        num = jnp.sum(not_ignore.astype(jnp.int32))
        tn_ref[0, 0] += num
        ts_ref[0, 0] += same_num
        tr_ref[0, 0] += eff


def kernel(decoder_out, ys_pad, ys_pad_lens, pred_acoustic_embeds, ignore_id, W):
    B, T = ys_pad.shape
    rcol = jnp.asarray(_R_CONST.reshape(B, T, 1))
    rrow = jnp.asarray(_R_CONST.reshape(B, 1, T))
    ys_i32 = ys_pad.astype(jnp.int32)
    ys3 = ys_i32.reshape(B, T, 1)
    lens = ys_pad_lens.astype(jnp.int32)
    ig = jnp.asarray(ignore_id, jnp.int32).reshape(1)

    tgt = jnp.arange(T, dtype=jnp.int32)[None, :] < lens[:, None]
    tgt3 = tgt.reshape(B, T, 1)
    idx = jnp.where(tgt, ys_i32, _V).reshape(-1)
    out2_flat = _sc_gather_fn()(W, idx)
    out2 = out2_flat.reshape(B, T, _D)

    out_shapes = (
        jax.ShapeDtypeStruct((B, T, _D), jnp.float32),
        jax.ShapeDtypeStruct((B, T, _D), jnp.float32),
        jax.ShapeDtypeStruct((1, 1), jnp.int32),
        jax.ShapeDtypeStruct((1, 1), jnp.int32),
        jax.ShapeDtypeStruct((1, 1), jnp.int32),
    )
    grid = (B // _ROWS,)
    smem = pltpu.SMEM
    in_specs = [
        pl.BlockSpec(memory_space=smem),
        pl.BlockSpec(memory_space=smem),
        pl.BlockSpec((_ROWS, T, _V), lambda g: (g, 0, 0)),
        pl.BlockSpec((_ROWS, T, 1), lambda g: (g, 0, 0)),
        pl.BlockSpec((_ROWS, T, _D), lambda g: (g, 0, 0)),
        pl.BlockSpec((_ROWS, T, _D), lambda g: (g, 0, 0)),
        pl.BlockSpec((_ROWS, T, 1), lambda g: (g, 0, 0)),
        pl.BlockSpec((_ROWS, 1, T), lambda g: (g, 0, 0)),
    ]
    out_specs = [
        pl.BlockSpec((_ROWS, T, _D), lambda g: (g, 0, 0)),
        pl.BlockSpec((_ROWS, T, _D), lambda g: (g, 0, 0)),
        pl.BlockSpec((1, 1), lambda g: (0, 0), memory_space=smem),
        pl.BlockSpec((1, 1), lambda g: (0, 0), memory_space=smem),
        pl.BlockSpec((1, 1), lambda g: (0, 0), memory_space=smem),
    ]
    o1, o3, tn, ts, tr = pl.pallas_call(
        _tc_body,
        grid=grid,
        in_specs=in_specs,
        out_specs=out_specs,
        out_shape=out_shapes,
    )(lens, ig, decoder_out, ys3, out2, pred_acoustic_embeds, rcol, rrow)

    return (o1, out2, o3, tgt3,
            tn.reshape(()), ts.reshape(()), tr.reshape(()))
